# baseline (speedup 1.0000x reference)
import functools

import jax
import jax.numpy as jnp
from jax import lax
from jax.experimental import pallas as pl
from jax.experimental.pallas import tpu as pltpu
from jax.experimental.pallas import tpu_sc as plsc

VOCAB = 1_000_000
EMBED = 64
ROWS = 4096
COLS = 200
NW = 32
QROWS = VOCAB // 2

NFULL = VOCAB // 128
TAIL_V0 = NFULL * 128
K1_ITERS = 123

_mesh = plsc.VectorSubcoreMesh(core_axis_name="c", subcore_axis_name="s")
_params = pltpu.CompilerParams(
    use_tc_tiling_on_sc=True, needs_layout_passes=False
)


@functools.partial(
    pl.kernel,
    mesh=_mesh,
    compiler_params=_params,
    out_type=jax.ShapeDtypeStruct((QROWS, 128), jnp.float32),
    scratch_types=[
        pltpu.VMEM((2, 64, 128), jnp.float32),
        pltpu.VMEM((2, 64, 128), jnp.float32),
        pltpu.SemaphoreType.DMA,
        pltpu.SemaphoreType.DMA,
        pltpu.SemaphoreType.DMA,
        pltpu.SemaphoreType.DMA,
    ],
)
def _repack(wt_hbm, qtail_hbm, q_hbm, vbuf, qbuf, si0, si1, so0, so1):
    wid = lax.axis_index("s") * 2 + lax.axis_index("c")
    n_w = jnp.where(wid < 4, 245, 244)
    iota = lax.iota(jnp.int32, 16)
    zero16 = iota * 0
    rowv = [iota + 16 * (j % 4) for j in range(8)]
    sis = (si0, si1)
    sos = (so0, so1)

    def v0_of(i):
        cid = wid + NW * i
        return pl.multiple_of(cid * 128, 128)

    def fire_in(i, slot):
        pltpu.async_copy(wt_hbm.at[:, pl.ds(v0_of(i), 128)], vbuf.at[slot], sis[slot])

    def wait_in(slot):
        pltpu.make_async_copy(
            wt_hbm.at[:, pl.ds(0, 128)], vbuf.at[slot], sis[slot]
        ).wait()

    def wait_out(slot):
        pltpu.make_async_copy(
            qbuf.at[slot], q_hbm.at[pl.ds(0, 64)], sos[slot]
        ).wait()

    fire_in(0, 0)

    def body(i2, carry):
        for sub in range(2):
            i = i2 * 2 + sub

            @pl.when(i + 1 < n_w)
            def _():
                fire_in(i + 1, 1 - sub)

            @pl.when(i < n_w)
            def _():
                wait_in(sub)

                @pl.when(i >= 2)
                def _():
                    wait_out(sub)

                def kkbody(kk0, c):
                    for kks in range(8):
                        kk = kk0 * 8 + kks
                        for j in range(8):
                            colv = zero16 + (2 * kk + j // 4)
                            v = plsc.load_gather(vbuf.at[sub], [rowv[j], colv])
                            qbuf[sub, kk, pl.ds(16 * j, 16)] = v
                    return c

                lax.fori_loop(0, 8, kkbody, 0)
                r0 = pl.multiple_of(lax.shift_right_logical(v0_of(i), 1), 64)
                pltpu.async_copy(qbuf.at[sub], q_hbm.at[pl.ds(r0, 64)], sos[sub])
        return carry

    lax.fori_loop(0, K1_ITERS, body, 0)
    wait_out(0)
    wait_out(1)

    @pl.when(wid == 4)
    def _():
        pltpu.sync_copy(qtail_hbm, q_hbm.at[pl.ds(QROWS - 32, 32)])


@functools.partial(
    pl.kernel,
    mesh=_mesh,
    compiler_params=_params,
    out_type=jax.ShapeDtypeStruct((COLS, 8, NW, 8, 128), jnp.float32),
    scratch_types=[
        pltpu.VMEM((8, 128), jnp.int32),
        pltpu.VMEM((2, 128), jnp.int32),
        pltpu.VMEM((2, 128), jnp.int32),
        pltpu.VMEM((2, 128, 128), jnp.float32),
        pltpu.VMEM((2, 64, 128), jnp.float32),
        pltpu.SemaphoreType.DMA,
        pltpu.SemaphoreType.DMA,
        pltpu.SemaphoreType.DMA,
        pltpu.SemaphoreType.DMA,
    ],
)
def _lookup(q_hbm, idxt_hbm, out_hbm, ibuf, pidx, cbase, pairbuf, obuf,
            sg0, sg1, so0, so1):
    wid = lax.axis_index("s") * 2 + lax.axis_index("c")
    b0 = wid * 128
    iota = lax.iota(jnp.int32, 16)
    sgs = (sg0, sg1)
    sos = (so0, so1)

    def load_idx_group(t):
        t8 = pl.multiple_of(t, 8)
        bb = pl.multiple_of(b0, 128)
        pltpu.sync_copy(idxt_hbm.at[pl.ds(t8, 8), pl.ds(bb, 128)], ibuf)

    def prep(t, slot):
        tt = lax.rem(t, 8)
        for j in range(8):
            iv = ibuf[tt, pl.ds(16 * j, 16)]
            pidx[slot, pl.ds(16 * j, 16)] = lax.shift_right_logical(iv, 1)
            cbase[slot, pl.ds(16 * j, 16)] = lax.shift_left(
                jnp.bitwise_and(iv, 1), 6
            )

    def fire_gather(slot):
        pltpu.async_copy(q_hbm.at[pidx.at[slot]], pairbuf.at[slot], sgs[slot])

    def wait_gather(slot):
        pltpu.make_async_copy(
            q_hbm.at[pidx.at[slot]], pairbuf.at[slot], sgs[slot]
        ).wait()

    def wait_out(slot):
        for _ in range(8):
            pltpu.make_async_copy(
                obuf.at[slot, pl.ds(0, 8)], out_hbm.at[0, 0, 0], sos[slot]
            ).wait()

    load_idx_group(0)
    prep(0, 0)
    fire_gather(0)

    def body(t2, carry):
        for sub in range(2):
            t = t2 * 2 + sub

            @pl.when(t + 1 < COLS)
            def _():
                @pl.when(lax.rem(t + 1, 8) == 0)
                def _():
                    load_idx_group(t + 1)

                prep(t + 1, 1 - sub)
                fire_gather(1 - sub)

            wait_gather(sub)

            @pl.when(t >= 2)
            def _():
                wait_out(sub)

            def ebody(e0, c):
                cbs = [cbase[sub, pl.ds(16 * j, 16)] for j in range(8)]
                for ee in range(8):
                    e = e0 * 8 + ee
                    for j in range(8):
                        v = plsc.load_gather(
                            pairbuf.at[sub], [iota + 16 * j, cbs[j] + e]
                        )
                        obuf[sub, e, pl.ds(16 * j, 16)] = v
                return c

            lax.fori_loop(0, 8, ebody, 0)
            for E in range(8):
                pltpu.async_copy(
                    obuf.at[sub, pl.ds(8 * E, 8)], out_hbm.at[t, E, wid], sos[sub]
                )
        return carry

    lax.fori_loop(0, COLS // 2, body, 0)
    wait_out(0)
    wait_out(1)


def kernel(indices, weight):
    wt = weight.T
    idxt = indices.T
    q_tail = weight[TAIL_V0:].reshape(32, 128)
    q = _repack(wt, q_tail)
    out5 = _lookup(q, idxt)
    return out5.transpose(2, 4, 0, 1, 3).reshape(ROWS, COLS, EMBED)


# speedup vs baseline: 1.8131x; 1.8131x over previous
import functools

import jax
import jax.numpy as jnp
from jax import lax
from jax.experimental import pallas as pl
from jax.experimental.pallas import tpu as pltpu
from jax.experimental.pallas import tpu_sc as plsc

VOCAB = 1_000_000
EMBED = 64
ROWS = 4096
COLS = 200
NW = 32
QROWS = VOCAB // 2

CW = 256
NFULL = VOCAB // CW
TAIL_V0 = NFULL * CW
K1_ITERS = 41

_mesh = plsc.VectorSubcoreMesh(core_axis_name="c", subcore_axis_name="s")


@functools.partial(
    pl.kernel,
    mesh=_mesh,
    compiler_params=pltpu.CompilerParams(
        use_tc_tiling_on_sc=True, needs_layout_passes=False
    ),
    out_type=jax.ShapeDtypeStruct((QROWS, 128), jnp.float32),
    scratch_types=[
        pltpu.VMEM((3, 64, CW), jnp.float32),
        pltpu.VMEM((3, 128, 128), jnp.float32),
        pltpu.SemaphoreType.DMA,
        pltpu.SemaphoreType.DMA,
        pltpu.SemaphoreType.DMA,
        pltpu.SemaphoreType.DMA,
        pltpu.SemaphoreType.DMA,
        pltpu.SemaphoreType.DMA,
    ],
)
def _repack(wt_hbm, qtail_hbm, q_hbm, vbuf, qbuf, si0, si1, si2, so0, so1, so2):
    wid = lax.axis_index("s") * 2 + lax.axis_index("c")
    n_w = jnp.where(wid < 2, 123, 122)
    iota = lax.iota(jnp.int32, 16)
    zero16 = iota * 0
    rowv = [iota + 16 * (j % 4) for j in range(8)]
    sis = (si0, si1, si2)
    sos = (so0, so1, so2)

    def v0_of(i):
        cid = wid + NW * i
        return pl.multiple_of(cid * CW, CW)

    def fire_in(i, slot):
        pltpu.async_copy(wt_hbm.at[:, pl.ds(v0_of(i), CW)], vbuf.at[slot], sis[slot])

    def wait_in(slot):
        pltpu.make_async_copy(
            wt_hbm.at[:, pl.ds(0, CW)], vbuf.at[slot], sis[slot]
        ).wait()

    def wait_out(slot):
        pltpu.make_async_copy(
            qbuf.at[slot], q_hbm.at[pl.ds(0, 128)], sos[slot]
        ).wait()

    fire_in(0, 0)
    fire_in(1, 1)

    def body(i3, carry):
        for sub in range(3):
            i = i3 * 3 + sub

            @pl.when(i + 2 < n_w)
            def _():
                fire_in(i + 2, (sub + 2) % 3)

            @pl.when(i < n_w)
            def _():
                wait_in(sub)

                @pl.when(i >= 3)
                def _():
                    wait_out(sub)

                @plsc.parallel_loop(0, 128, 1, unroll=8)
                def _(kk):
                    for j in range(8):
                        colv = zero16 + (2 * kk + j // 4)
                        v = plsc.load_gather(vbuf.at[sub], [rowv[j], colv])
                        qbuf[sub, kk, pl.ds(16 * j, 16)] = v

                r0 = pl.multiple_of(lax.shift_right_logical(v0_of(i), 1), 128)
                pltpu.async_copy(qbuf.at[sub], q_hbm.at[pl.ds(r0, 128)], sos[sub])
        return carry

    lax.fori_loop(0, K1_ITERS, body, 0)
    wait_out(0)
    wait_out(1)
    wait_out(2)

    @pl.when(wid == 4)
    def _():
        pltpu.sync_copy(qtail_hbm, q_hbm.at[pl.ds(QROWS - 32, 32)])


@functools.partial(
    pl.kernel,
    mesh=_mesh,
    compiler_params=pltpu.CompilerParams(
        use_tc_tiling_on_sc=False, needs_layout_passes=False
    ),
    out_type=jax.ShapeDtypeStruct((COLS, 8, NW, 8, 128), jnp.float32),
    scratch_types=[
        pltpu.VMEM((2, 8, 128), jnp.int32),
        pltpu.VMEM((4, 128), jnp.int32),
        pltpu.VMEM((4, 128, EMBED), jnp.float32),
        pltpu.VMEM((2, 64, 128), jnp.float32),
        pltpu.SemaphoreType.DMA,
        pltpu.SemaphoreType.DMA,
        pltpu.SemaphoreType.DMA,
        pltpu.SemaphoreType.DMA,
        pltpu.SemaphoreType.DMA,
        pltpu.SemaphoreType.DMA,
    ],
)
def _lookup(q_hbm, idxt_hbm, out_hbm, ibuf, pidx, rbuf, obuf,
            sg0, sg1, sg2, sg3, so0, so1):
    wid = lax.axis_index("s") * 2 + lax.axis_index("c")
    b0 = wid * 128
    iota = lax.iota(jnp.int32, 16)
    zero16 = iota * 0
    rows = [iota + 16 * j for j in range(8)]
    sgs = (sg0, sg1, sg2, sg3)
    sos = (so0, so1)

    def load_idx_group(t):
        g2 = lax.rem(lax.div(t, 8), 2)
        pltpu.sync_copy(
            idxt_hbm.at[pl.ds(pl.multiple_of(t, 8), 8), pl.ds(b0, 128)],
            ibuf.at[g2],
        )

    def prep(t, slot):
        g2 = lax.rem(lax.div(t, 8), 2)
        tt = lax.rem(t, 8)
        for j in range(8):
            pidx[slot, pl.ds(16 * j, 16)] = ibuf[g2, tt, pl.ds(16 * j, 16)]

    def fire_gather(slot):
        pltpu.async_copy(q_hbm.at[pidx.at[slot]], rbuf.at[slot], sgs[slot])

    def wait_gather(slot):
        pltpu.make_async_copy(
            q_hbm.at[pidx.at[slot]], rbuf.at[slot], sgs[slot]
        ).wait()

    def wait_out(slot):
        for _ in range(8):
            pltpu.make_async_copy(
                obuf.at[slot, pl.ds(0, 8)], out_hbm.at[0, 0, 0], sos[slot]
            ).wait()

    load_idx_group(0)
    for tp in range(3):
        prep(tp, tp)
        fire_gather(tp)

    def body(t4, carry):
        for sub in range(4):
            t = t4 * 4 + sub

            @pl.when(t + 3 < COLS)
            def _():
                @pl.when(lax.rem(t + 3, 8) == 0)
                def _():
                    load_idx_group(t + 3)

                prep(t + 3, (sub + 3) % 4)
                fire_gather((sub + 3) % 4)

            wait_gather(sub)

            @pl.when(t >= 2)
            def _():
                wait_out(sub % 2)

            @plsc.parallel_loop(0, 64, 1, unroll=8)
            def _(e):
                colv = zero16 + e
                for j in range(8):
                    v = plsc.load_gather(rbuf.at[sub], [rows[j], colv])
                    obuf[sub % 2, e, pl.ds(16 * j, 16)] = v

            for E in range(8):
                pltpu.async_copy(
                    obuf.at[sub % 2, pl.ds(8 * E, 8)],
                    out_hbm.at[t, E, wid],
                    sos[sub % 2],
                )
        return carry

    lax.fori_loop(0, COLS // 4, body, 0)
    wait_out(0)
    wait_out(1)


def kernel(indices, weight):
    wt = weight.T
    idxt = indices.T
    q_tail = weight[TAIL_V0:].reshape(32, 128)
    q = _repack(wt, q_tail)
    qlin = q.reshape(VOCAB, EMBED)
    out5 = _lookup(qlin, idxt)
    return out5.transpose(2, 4, 0, 1, 3).reshape(ROWS, COLS, EMBED)


# speedup vs baseline: 1.8242x; 1.0061x over previous
import functools

import jax
import jax.numpy as jnp
from jax import lax
from jax.experimental import pallas as pl
from jax.experimental.pallas import tpu as pltpu
from jax.experimental.pallas import tpu_sc as plsc

VOCAB = 1_000_000
EMBED = 64
ROWS = 4096
COLS = 200
NW = 32
QROWS = VOCAB // 2

CW = 256
NFULL = VOCAB // CW
TAIL_V0 = NFULL * CW
K1_ITERS = 31

GT = 4
NG = COLS // GT

_mesh = plsc.VectorSubcoreMesh(core_axis_name="c", subcore_axis_name="s")


@functools.partial(
    pl.kernel,
    mesh=_mesh,
    compiler_params=pltpu.CompilerParams(
        use_tc_tiling_on_sc=True, needs_layout_passes=False
    ),
    out_type=jax.ShapeDtypeStruct((QROWS, 128), jnp.float32),
    scratch_types=[
        pltpu.VMEM((4, 8, 8, CW), jnp.float32),
        pltpu.VMEM((2, CW // 2, 128), jnp.float32),
        pltpu.SemaphoreType.DMA,
        pltpu.SemaphoreType.DMA,
        pltpu.SemaphoreType.DMA,
        pltpu.SemaphoreType.DMA,
        pltpu.SemaphoreType.DMA,
        pltpu.SemaphoreType.DMA,
    ],
)
def _repack(wt_hbm, qtail_hbm, q_hbm, sbuf, qbuf, si0, si1, si2, si3, so0, so1):
    wid = lax.axis_index("s") * 2 + lax.axis_index("c")
    n_w = jnp.where(wid < 2, 123, 122)
    iota = lax.iota(jnp.int32, 16)
    zero16 = iota * 0
    gv = [2 * (j % 4) + lax.div(iota, 8) for j in range(4)]
    erv = lax.rem(iota, 8)
    sis = (si0, si1, si2, si3)
    sos = (so0, so1)

    def v0_of(i):
        cid = wid + NW * i
        return pl.multiple_of(cid * CW, CW)

    def fire_in(i, slot):
        v0 = v0_of(i)
        for g in range(8):
            pltpu.async_copy(
                wt_hbm.at[pl.ds(8 * g, 8), pl.ds(v0, CW)],
                sbuf.at[slot, g],
                sis[slot],
            )

    def wait_in(slot):
        for _ in range(8):
            pltpu.make_async_copy(
                wt_hbm.at[pl.ds(0, 8), pl.ds(0, CW)], sbuf.at[slot, 0], sis[slot]
            ).wait()

    def wait_out(slot):
        pltpu.make_async_copy(
            qbuf.at[slot], q_hbm.at[pl.ds(0, CW // 2)], sos[slot]
        ).wait()

    fire_in(0, 0)
    fire_in(1, 1)
    fire_in(2, 2)

    def body(i4, carry):
        for sub in range(4):
            i = i4 * 4 + sub

            @pl.when(i + 3 < n_w)
            def _():
                fire_in(i + 3, (sub + 3) % 4)

            @pl.when(i < n_w)
            def _():
                wait_in(sub)

                @pl.when(i >= 2)
                def _():
                    wait_out(sub % 2)

                @plsc.parallel_loop(0, CW // 2, 1, unroll=8)
                def _(kk):
                    for j in range(8):
                        vvv = zero16 + (2 * kk + j // 4)
                        v = plsc.load_gather(
                            sbuf.at[sub], [gv[j % 4], erv, vvv]
                        )
                        qbuf[sub % 2, kk, pl.ds(16 * j, 16)] = v

                r0 = pl.multiple_of(
                    lax.shift_right_logical(v0_of(i), 1), CW // 2
                )
                pltpu.async_copy(
                    qbuf.at[sub % 2], q_hbm.at[pl.ds(r0, CW // 2)], sos[sub % 2]
                )
        return carry

    lax.fori_loop(0, K1_ITERS, body, 0)
    wait_out(0)
    wait_out(1)

    @pl.when(wid == 4)
    def _():
        pltpu.sync_copy(qtail_hbm, q_hbm.at[pl.ds(QROWS - 32, 32)])


@functools.partial(
    pl.kernel,
    mesh=_mesh,
    compiler_params=pltpu.CompilerParams(
        use_tc_tiling_on_sc=False, needs_layout_passes=False
    ),
    out_type=jax.ShapeDtypeStruct((COLS, 8, NW, 8, 128), jnp.float32),
    scratch_types=[
        pltpu.VMEM((2, 8, 128), jnp.int32),
        pltpu.VMEM((2, GT * 128), jnp.int32),
        pltpu.VMEM((2, GT * 128, EMBED), jnp.float32),
        pltpu.VMEM((2, 8, 8, 128), jnp.float32),
        pltpu.SemaphoreType.DMA,
        pltpu.SemaphoreType.DMA,
        pltpu.SemaphoreType.DMA,
        pltpu.SemaphoreType.DMA,
    ],
)
def _lookup(q_hbm, idxt_hbm, out_hbm, ibuf, pidx, rbuf, obuf, sg0, sg1, so0, so1):
    wid = lax.axis_index("s") * 2 + lax.axis_index("c")
    b0 = wid * 128
    iota = lax.iota(jnp.int32, 16)
    zero16 = iota * 0
    rows = [iota + 16 * j for j in range(8)]
    sgs = (sg0, sg1)
    sos = (so0, so1)

    def load_idx_group(t):
        g2 = lax.rem(lax.div(t, 8), 2)
        pltpu.sync_copy(
            idxt_hbm.at[pl.ds(pl.multiple_of(t, 8), 8), pl.ds(b0, 128)],
            ibuf.at[g2],
        )

    def prep_group(g, slot):
        g2 = lax.rem(lax.div(g, 2), 2)
        tbase = 4 * lax.rem(g, 2)
        for r in range(GT):
            for j in range(8):
                pidx[slot, pl.ds(r * 128 + 16 * j, 16)] = ibuf[
                    g2, tbase + r, pl.ds(16 * j, 16)
                ]

    def fire_gather(slot):
        pltpu.async_copy(q_hbm.at[pidx.at[slot]], rbuf.at[slot], sgs[slot])

    def wait_gather(slot):
        pltpu.make_async_copy(
            q_hbm.at[pidx.at[slot]], rbuf.at[slot], sgs[slot]
        ).wait()

    def wait_out(slot):
        pltpu.make_async_copy(
            obuf.at[slot], out_hbm.at[0, :, 0], sos[slot]
        ).wait()

    load_idx_group(0)
    prep_group(0, 0)
    fire_gather(0)

    def body(gc, carry):
        for sub in range(2):
            g = gc * 2 + sub

            @pl.when(g + 1 < NG)
            def _():
                @pl.when(lax.rem(g + 1, 2) == 0)
                def _():
                    load_idx_group(4 * (g + 1))

                prep_group(g + 1, 1 - sub)
                fire_gather(1 - sub)

            wait_gather(sub)

            for r in range(GT):
                t = 4 * g + r

                @pl.when(t >= 2)
                def _():
                    wait_out(r % 2)

                @plsc.parallel_loop(0, 64, 1, unroll=8)
                def _(e):
                    colv = zero16 + e
                    E = lax.div(e, 8)
                    s = lax.rem(e, 8)
                    for j in range(8):
                        v = plsc.load_gather(
                            rbuf.at[sub, pl.ds(r * 128, 128)], [rows[j], colv]
                        )
                        obuf[r % 2, E, s, pl.ds(16 * j, 16)] = v

                pltpu.async_copy(
                    obuf.at[r % 2], out_hbm.at[t, :, wid], sos[r % 2]
                )
        return carry

    lax.fori_loop(0, NG // 2, body, 0)
    wait_out(0)
    wait_out(1)


def kernel(indices, weight):
    wt = weight.T
    idxt = indices.T
    q_tail = weight[TAIL_V0:].reshape(32, 128)
    q = _repack(wt, q_tail)
    qlin = q.reshape(VOCAB, EMBED)
    out5 = _lookup(qlin, idxt)
    return out5.transpose(2, 4, 0, 1, 3).reshape(ROWS, COLS, EMBED)


# speedup vs baseline: 4.4260x; 2.4263x over previous
import functools

import jax
import jax.numpy as jnp
from jax import lax
from jax.experimental import pallas as pl
from jax.experimental.pallas import tpu as pltpu
from jax.experimental.pallas import tpu_sc as plsc

VOCAB = 1_000_000
EMBED = 64
ROWS = 4096
COLS = 200
NW = 32
QROWS = VOCAB // 2

CW = 256
NFULL = VOCAB // CW
TAIL_V0 = NFULL * CW
K1_ITERS = 31

GT = 4
NG = COLS // GT

_mesh = plsc.VectorSubcoreMesh(core_axis_name="c", subcore_axis_name="s")


@functools.partial(
    pl.kernel,
    mesh=_mesh,
    compiler_params=pltpu.CompilerParams(
        use_tc_tiling_on_sc=True, needs_layout_passes=False
    ),
    out_type=jax.ShapeDtypeStruct((QROWS, 128), jnp.float32),
    scratch_types=[
        pltpu.VMEM((4, 8, 8, CW), jnp.float32),
        pltpu.VMEM((2, CW // 2, 128), jnp.float32),
        pltpu.SemaphoreType.DMA,
        pltpu.SemaphoreType.DMA,
        pltpu.SemaphoreType.DMA,
        pltpu.SemaphoreType.DMA,
        pltpu.SemaphoreType.DMA,
        pltpu.SemaphoreType.DMA,
    ],
)
def _repack(wt_hbm, qtail_hbm, q_hbm, sbuf, qbuf, si0, si1, si2, si3, so0, so1):
    wid = lax.axis_index("s") * 2 + lax.axis_index("c")
    n_w = jnp.where(wid < 2, 123, 122)
    iota = lax.iota(jnp.int32, 16)
    hio = lax.shift_right_logical(iota, 1)
    par64 = lax.shift_left(jnp.bitwise_and(iota, 1), 6)
    sis = (si0, si1, si2, si3)
    sos = (so0, so1)

    def v0_of(i):
        cid = wid + NW * i
        return pl.multiple_of(cid * CW, CW)

    def fire_in(i, slot):
        v0 = v0_of(i)
        for g in range(8):
            pltpu.async_copy(
                wt_hbm.at[pl.ds(8 * g, 8), pl.ds(v0, CW)],
                sbuf.at[slot, g],
                sis[slot],
            )

    def wait_in(slot):
        for _ in range(8):
            pltpu.make_async_copy(
                wt_hbm.at[pl.ds(0, 8), pl.ds(0, CW)], sbuf.at[slot, 0], sis[slot]
            ).wait()

    def wait_out(slot):
        pltpu.make_async_copy(
            qbuf.at[slot], q_hbm.at[pl.ds(0, CW // 2)], sos[slot]
        ).wait()

    fire_in(0, 0)
    fire_in(1, 1)
    fire_in(2, 2)

    def body(i4, carry):
        for sub in range(4):
            i = i4 * 4 + sub

            @pl.when(i + 3 < n_w)
            def _():
                fire_in(i + 3, (sub + 3) % 4)

            @pl.when(i < n_w)
            def _():
                wait_in(sub)

                @pl.when(i >= 2)
                def _():
                    wait_out(sub % 2)

                @plsc.parallel_loop(0, 16, 1, unroll=2)
                def _(k):
                    rv = lax.rem(iota + k, 16)
                    gvv = lax.shift_right_logical(rv, 3)
                    erv = jnp.bitwise_and(rv, 7)
                    cpar = rv + par64
                    for e0 in (0, 16, 32, 48):
                        gvec = gvv + (e0 // 8)
                        colq = cpar + e0
                        for m in range(16):
                            vvv = iota + 16 * m
                            rowq = hio + 8 * m
                            v = plsc.load_gather(
                                sbuf.at[sub], [gvec, erv, vvv]
                            )
                            plsc.store_scatter(
                                qbuf.at[sub % 2], [rowq, colq], v
                            )

                r0 = pl.multiple_of(
                    lax.shift_right_logical(v0_of(i), 1), CW // 2
                )
                pltpu.async_copy(
                    qbuf.at[sub % 2], q_hbm.at[pl.ds(r0, CW // 2)], sos[sub % 2]
                )
        return carry

    lax.fori_loop(0, K1_ITERS, body, 0)
    wait_out(0)
    wait_out(1)

    @pl.when(wid == 4)
    def _():
        pltpu.sync_copy(qtail_hbm, q_hbm.at[pl.ds(QROWS - 32, 32)])


@functools.partial(
    pl.kernel,
    mesh=_mesh,
    compiler_params=pltpu.CompilerParams(
        use_tc_tiling_on_sc=False, needs_layout_passes=False
    ),
    out_type=jax.ShapeDtypeStruct((COLS, 8, NW, 8, 128), jnp.float32),
    scratch_types=[
        pltpu.VMEM((2, 8, 128), jnp.int32),
        pltpu.VMEM((2, GT * 128), jnp.int32),
        pltpu.VMEM((2, GT * 128, EMBED), jnp.float32),
        pltpu.VMEM((2, 8, 8, 128), jnp.float32),
        pltpu.SemaphoreType.DMA,
        pltpu.SemaphoreType.DMA,
        pltpu.SemaphoreType.DMA,
        pltpu.SemaphoreType.DMA,
    ],
)
def _lookup(q_hbm, idxt_hbm, out_hbm, ibuf, pidx, rbuf, obuf, sg0, sg1, so0, so1):
    wid = lax.axis_index("s") * 2 + lax.axis_index("c")
    b0 = wid * 128
    iota = lax.iota(jnp.int32, 16)
    rows = [iota + 16 * j for j in range(8)]
    sgs = (sg0, sg1)
    sos = (so0, so1)

    def load_idx_group(t):
        g2 = lax.rem(lax.div(t, 8), 2)
        pltpu.sync_copy(
            idxt_hbm.at[pl.ds(pl.multiple_of(t, 8), 8), pl.ds(b0, 128)],
            ibuf.at[g2],
        )

    def prep_group(g, slot):
        g2 = lax.rem(lax.div(g, 2), 2)
        tbase = 4 * lax.rem(g, 2)
        for r in range(GT):
            for j in range(8):
                pidx[slot, pl.ds(r * 128 + 16 * j, 16)] = ibuf[
                    g2, tbase + r, pl.ds(16 * j, 16)
                ]

    def fire_gather(slot):
        pltpu.async_copy(q_hbm.at[pidx.at[slot]], rbuf.at[slot], sgs[slot])

    def wait_gather(slot):
        pltpu.make_async_copy(
            q_hbm.at[pidx.at[slot]], rbuf.at[slot], sgs[slot]
        ).wait()

    def wait_out(slot):
        pltpu.make_async_copy(
            obuf.at[slot], out_hbm.at[0, :, 0], sos[slot]
        ).wait()

    load_idx_group(0)
    prep_group(0, 0)
    fire_gather(0)

    def body(gc, carry):
        for sub in range(2):
            g = gc * 2 + sub

            @pl.when(g + 1 < NG)
            def _():
                @pl.when(lax.rem(g + 1, 2) == 0)
                def _():
                    load_idx_group(4 * (g + 1))

                prep_group(g + 1, 1 - sub)
                fire_gather(1 - sub)

            wait_gather(sub)

            for r in range(GT):
                t = 4 * g + r

                @pl.when(t >= 2)
                def _():
                    wait_out(r % 2)

                @plsc.parallel_loop(0, 16, 1, unroll=2)
                def _(k):
                    rv = lax.rem(iota + k, 16)
                    for e0 in (0, 16, 32, 48):
                        erowv = rv + e0
                        gE = lax.shift_right_logical(erowv, 3)
                        gs = jnp.bitwise_and(erowv, 7)
                        for j in range(8):
                            v = plsc.load_gather(
                                rbuf.at[sub, pl.ds(r * 128, 128)],
                                [rows[j], erowv],
                            )
                            plsc.store_scatter(
                                obuf.at[r % 2], [gE, gs, rows[j]], v
                            )

                pltpu.async_copy(
                    obuf.at[r % 2], out_hbm.at[t, :, wid], sos[r % 2]
                )
        return carry

    lax.fori_loop(0, NG // 2, body, 0)
    wait_out(0)
    wait_out(1)


def kernel(indices, weight):
    wt = weight.T
    idxt = indices.T
    q_tail = weight[TAIL_V0:].reshape(32, 128)
    q = _repack(wt, q_tail)
    qlin = q.reshape(VOCAB, EMBED)
    out5 = _lookup(qlin, idxt)
    return out5.transpose(2, 4, 0, 1, 3).reshape(ROWS, COLS, EMBED)


# speedup vs baseline: 4.4821x; 1.0127x over previous
import functools

import jax
import jax.numpy as jnp
from jax import lax
from jax.experimental import pallas as pl
from jax.experimental.pallas import tpu as pltpu
from jax.experimental.pallas import tpu_sc as plsc

VOCAB = 1_000_000
EMBED = 64
ROWS = 4096
COLS = 200
NW = 32
QROWS = VOCAB // 2

CW = 256
NFULL = VOCAB // CW
TAIL_V0 = NFULL * CW
K1_ITERS = 31

GT = 4
NG = COLS // GT

_mesh = plsc.VectorSubcoreMesh(core_axis_name="c", subcore_axis_name="s")


@functools.partial(
    pl.kernel,
    mesh=_mesh,
    compiler_params=pltpu.CompilerParams(
        use_tc_tiling_on_sc=True, needs_layout_passes=False
    ),
    out_type=jax.ShapeDtypeStruct((QROWS, 128), jnp.float32),
    scratch_types=[
        pltpu.VMEM((4, 64, CW), jnp.float32),
        pltpu.VMEM((2, CW // 2, 128), jnp.float32),
        pltpu.SemaphoreType.DMA,
        pltpu.SemaphoreType.DMA,
        pltpu.SemaphoreType.DMA,
        pltpu.SemaphoreType.DMA,
        pltpu.SemaphoreType.DMA,
        pltpu.SemaphoreType.DMA,
    ],
)
def _repack(wt_hbm, qtail_hbm, q_hbm, sbuf, qbuf, si0, si1, si2, si3, so0, so1):
    wid = lax.axis_index("s") * 2 + lax.axis_index("c")
    n_w = jnp.where(wid < 2, 123, 122)
    iota = lax.iota(jnp.int32, 16)
    hio = lax.shift_right_logical(iota, 1)
    par64 = lax.shift_left(jnp.bitwise_and(iota, 1), 6)
    sis = (si0, si1, si2, si3)
    sos = (so0, so1)

    def v0_of(i):
        cid = wid + NW * i
        return pl.multiple_of(cid * CW, CW)

    def fire_in(i, slot):
        pltpu.async_copy(
            wt_hbm.at[:, pl.ds(v0_of(i), CW)], sbuf.at[slot], sis[slot]
        )

    def wait_in(slot):
        pltpu.make_async_copy(
            wt_hbm.at[:, pl.ds(0, CW)], sbuf.at[slot], sis[slot]
        ).wait()

    def wait_out(slot):
        pltpu.make_async_copy(
            qbuf.at[slot], q_hbm.at[pl.ds(0, CW // 2)], sos[slot]
        ).wait()

    fire_in(0, 0)
    fire_in(1, 1)
    fire_in(2, 2)

    def body(i4, carry):
        for sub in range(4):
            i = i4 * 4 + sub

            @pl.when(i + 3 < n_w)
            def _():
                fire_in(i + 3, (sub + 3) % 4)

            @pl.when(i < n_w)
            def _():
                wait_in(sub)

                @pl.when(i >= 2)
                def _():
                    wait_out(sub % 2)

                @plsc.parallel_loop(0, 16, 1, unroll=2)
                def _(k):
                    rv = lax.rem(iota + k, 16)
                    cpar = rv + par64
                    for e0 in (0, 16, 32, 48):
                        erowv = rv + e0
                        colq = cpar + e0
                        for m in range(CW // 16):
                            vvv = iota + 16 * m
                            rowq = hio + 8 * m
                            v = plsc.load_gather(
                                sbuf.at[sub], [erowv, vvv]
                            )
                            plsc.store_scatter(
                                qbuf.at[sub % 2], [rowq, colq], v
                            )

                r0 = pl.multiple_of(
                    lax.shift_right_logical(v0_of(i), 1), CW // 2
                )
                pltpu.async_copy(
                    qbuf.at[sub % 2], q_hbm.at[pl.ds(r0, CW // 2)], sos[sub % 2]
                )
        return carry

    lax.fori_loop(0, K1_ITERS, body, 0)
    wait_out(0)
    wait_out(1)

    @pl.when(wid == 4)
    def _():
        pltpu.sync_copy(qtail_hbm, q_hbm.at[pl.ds(QROWS - 32, 32)])


@functools.partial(
    pl.kernel,
    mesh=_mesh,
    compiler_params=pltpu.CompilerParams(
        use_tc_tiling_on_sc=False, needs_layout_passes=False
    ),
    out_type=jax.ShapeDtypeStruct((COLS, 8, NW, 8, 128), jnp.float32),
    scratch_types=[
        pltpu.VMEM((2, 8, 128), jnp.int32),
        pltpu.VMEM((2, GT * 128), jnp.int32),
        pltpu.VMEM((2, GT * 128, EMBED), jnp.float32),
        pltpu.VMEM((2, 8, 8, 128), jnp.float32),
        pltpu.SemaphoreType.DMA,
        pltpu.SemaphoreType.DMA,
        pltpu.SemaphoreType.DMA,
        pltpu.SemaphoreType.DMA,
    ],
)
def _lookup(q_hbm, idxt_hbm, out_hbm, ibuf, pidx, rbuf, obuf, sg0, sg1, so0, so1):
    wid = lax.axis_index("s") * 2 + lax.axis_index("c")
    b0 = wid * 128
    iota = lax.iota(jnp.int32, 16)
    rows = [iota + 16 * j for j in range(8)]
    sgs = (sg0, sg1)
    sos = (so0, so1)

    def load_idx_group(t):
        g2 = lax.rem(lax.div(t, 8), 2)
        pltpu.sync_copy(
            idxt_hbm.at[pl.ds(pl.multiple_of(t, 8), 8), pl.ds(b0, 128)],
            ibuf.at[g2],
        )

    def prep_group(g, slot):
        g2 = lax.rem(lax.div(g, 2), 2)
        tbase = 4 * lax.rem(g, 2)
        for r in range(GT):
            for j in range(8):
                pidx[slot, pl.ds(r * 128 + 16 * j, 16)] = ibuf[
                    g2, tbase + r, pl.ds(16 * j, 16)
                ]

    def fire_gather(slot):
        pltpu.async_copy(q_hbm.at[pidx.at[slot]], rbuf.at[slot], sgs[slot])

    def wait_gather(slot):
        pltpu.make_async_copy(
            q_hbm.at[pidx.at[slot]], rbuf.at[slot], sgs[slot]
        ).wait()

    def wait_out(slot):
        pltpu.make_async_copy(
            obuf.at[slot], out_hbm.at[0, :, 0], sos[slot]
        ).wait()

    load_idx_group(0)
    prep_group(0, 0)
    fire_gather(0)

    def body(gc, carry):
        for sub in range(2):
            g = gc * 2 + sub

            @pl.when(g + 1 < NG)
            def _():
                @pl.when(lax.rem(g + 1, 2) == 0)
                def _():
                    load_idx_group(4 * (g + 1))

                prep_group(g + 1, 1 - sub)
                fire_gather(1 - sub)

            wait_gather(sub)

            for r in range(GT):
                t = 4 * g + r

                @pl.when(t >= 2)
                def _():
                    wait_out(r % 2)

                @plsc.parallel_loop(0, 16, 1, unroll=2)
                def _(k):
                    rv = lax.rem(iota + k, 16)
                    for e0 in (0, 16, 32, 48):
                        erowv = rv + e0
                        gE = lax.shift_right_logical(erowv, 3)
                        gs = jnp.bitwise_and(erowv, 7)
                        for j in range(8):
                            v = plsc.load_gather(
                                rbuf.at[sub, pl.ds(r * 128, 128)],
                                [rows[j], erowv],
                            )
                            plsc.store_scatter(
                                obuf.at[r % 2], [gE, gs, rows[j]], v
                            )

                pltpu.async_copy(
                    obuf.at[r % 2], out_hbm.at[t, :, wid], sos[r % 2]
                )
        return carry

    lax.fori_loop(0, NG // 2, body, 0)
    wait_out(0)
    wait_out(1)


def kernel(indices, weight):
    wt = weight.T
    idxt = indices.T
    q_tail = weight[TAIL_V0:].reshape(32, 128)
    q = _repack(wt, q_tail)
    qlin = q.reshape(VOCAB, EMBED)
    out5 = _lookup(qlin, idxt)
    return out5.transpose(2, 4, 0, 1, 3).reshape(ROWS, COLS, EMBED)
